# baseline (speedup 1.0000x reference)
import functools

import jax
import jax.numpy as jnp
from jax import lax
from jax.experimental import pallas as pl
from jax.experimental.pallas import tpu as pltpu
from jax.experimental.pallas import tpu_sc as plsc

BATCH = 16384
EMBED_DIM = 64

_NC, _NS = 2, 16
_NW = _NC * _NS
_B_PER_W = BATCH // _NW
_CHUNK = 128
_NCHUNK = _B_PER_W // _CHUNK


def _gather_body(lt_hbm, ct_hbm, li_hbm, ci_hbm, el_hbm, ec_hbm,
                 li_v, ci_v, el_v, ec_v, sem):
    wid = lax.axis_index("s") * _NC + lax.axis_index("c")
    base = wid * _B_PER_W
    pltpu.sync_copy(li_hbm.at[pl.ds(base, _B_PER_W)], li_v)
    pltpu.sync_copy(ci_hbm.at[pl.ds(base, _B_PER_W)], ci_v)
    copies = []
    for j in range(_NCHUNK):
        sl = pl.ds(j * _CHUNK, _CHUNK)
        copies.append(pltpu.async_copy(lt_hbm.at[li_v.at[sl]], el_v.at[sl], sem))
        copies.append(pltpu.async_copy(ct_hbm.at[ci_v.at[sl]], ec_v.at[sl], sem))
    for c in copies:
        c.wait()
    pltpu.sync_copy(el_v, el_hbm.at[pl.ds(base, _B_PER_W)])
    pltpu.sync_copy(ec_v, ec_hbm.at[pl.ds(base, _B_PER_W)])


@jax.jit
def _sc_gather(label_table, cat_table, label_idx, category_idx):
    mesh = plsc.VectorSubcoreMesh(core_axis_name="c", subcore_axis_name="s")
    out_type = [
        jax.ShapeDtypeStruct((BATCH, EMBED_DIM), jnp.float32),
        jax.ShapeDtypeStruct((BATCH, EMBED_DIM), jnp.float32),
    ]
    scratch = [
        pltpu.VMEM((_B_PER_W,), jnp.int32),
        pltpu.VMEM((_B_PER_W,), jnp.int32),
        pltpu.VMEM((_B_PER_W, EMBED_DIM), jnp.float32),
        pltpu.VMEM((_B_PER_W, EMBED_DIM), jnp.float32),
        pltpu.SemaphoreType.DMA,
    ]
    fn = pl.kernel(_gather_body, out_type=out_type, mesh=mesh,
                   scratch_types=scratch,
                   compiler_params=pltpu.CompilerParams(
                       use_tc_tiling_on_sc=False))
    return fn(label_table, cat_table, label_idx, category_idx)


def _mlp_body(x1_ref, x2_ref, w1a_ref, w1b_ref, b1_ref, w2_ref, b2_ref,
              w3_ref, b3_ref, o_ref):
    x1 = x1_ref[...]
    x2 = x2_ref[...]
    h = jnp.dot(x1, w1a_ref[...], preferred_element_type=jnp.float32)
    h += jnp.dot(x2, w1b_ref[...], preferred_element_type=jnp.float32)
    h = jnp.maximum(h + b1_ref[...], 0.0)
    h = jnp.dot(h, w2_ref[...], preferred_element_type=jnp.float32)
    h = jnp.maximum(h + b2_ref[...], 0.0)
    o_ref[...] = jnp.dot(h, w3_ref[...],
                         preferred_element_type=jnp.float32) + b3_ref[...]


@functools.partial(jax.jit, static_argnames=("bm",))
def _tc_mlp(e_label, e_cat, W1a, W1b, b1, W2, b2, W3, b3, bm=2048):
    grid = (BATCH // bm,)
    full = lambda shape: pl.BlockSpec(shape, lambda i: (0, 0))
    return pl.pallas_call(
        _mlp_body,
        grid=grid,
        in_specs=[
            pl.BlockSpec((bm, EMBED_DIM), lambda i: (i, 0)),
            pl.BlockSpec((bm, EMBED_DIM), lambda i: (i, 0)),
            full(W1a.shape),
            full(W1b.shape),
            full(b1.shape),
            full(W2.shape),
            full(b2.shape),
            full(W3.shape),
            full(b3.shape),
        ],
        out_specs=pl.BlockSpec((bm, 2), lambda i: (i, 0)),
        out_shape=jax.ShapeDtypeStruct((BATCH, 2), jnp.float32),
    )(e_label, e_cat, W1a, W1b, b1, W2, b2, W3, b3)


def kernel(label_idx, category_idx, label_table, cat_table,
           W1, b1, W2, b2, W3, b3):
    li = label_idx.astype(jnp.int32)
    ci = category_idx.astype(jnp.int32)
    e_label, e_cat = _sc_gather(label_table, cat_table, li, ci)
    W1a = W1[:EMBED_DIM]
    W1b = W1[EMBED_DIM:]
    return _tc_mlp(e_label, e_cat, W1a, W1b, b1.reshape(1, -1), W2,
                   b2.reshape(1, -1), W3, b3.reshape(1, -1))


# speedup vs baseline: 1.6493x; 1.6493x over previous
import functools

import jax
import jax.numpy as jnp
from jax import lax
from jax.experimental import pallas as pl
from jax.experimental.pallas import tpu as pltpu
from jax.experimental.pallas import tpu_sc as plsc

BATCH = 16384
EMBED_DIM = 64

_NC, _NS = 2, 16
_NW = _NC * _NS
_B_PER_W = BATCH // _NW
_CHUNK = 256
_NCHUNK = _B_PER_W // _CHUNK


_LANES = 16
_ONEHOT = [None] * _LANES


def _extract(vec, j):
    lane = lax.broadcasted_iota(jnp.int32, (_LANES,), 0)
    masked = jnp.where(lane == j, vec, jnp.int32(0))
    return jnp.max(masked)


def _gather_body(lt_hbm, ct_hbm, li_hbm, ci_hbm, el_hbm, ec_hbm,
                 li_v, ci_v, el_v, ec_v, sem):
    wid = lax.axis_index("s") * _NC + lax.axis_index("c")
    base = wid * _B_PER_W
    pltpu.sync_copy(li_hbm.at[pl.ds(base, _B_PER_W)], li_v)
    pltpu.sync_copy(ci_hbm.at[pl.ds(base, _B_PER_W)], ci_v)
    for c in range(_NCHUNK):
        off = c * _CHUNK

        def issue(g, _):
            lv = li_v[pl.ds(off + g * _LANES, _LANES)]
            cv = ci_v[pl.ds(off + g * _LANES, _LANES)]
            for j in range(_LANES):
                li = _extract(lv, j)
                ci = _extract(cv, j)
                i = g * _LANES + j
                pltpu.make_async_copy(lt_hbm.at[pl.ds(li, 1), :],
                                      el_v.at[pl.ds(i, 1), :], sem).start()
                pltpu.make_async_copy(ct_hbm.at[pl.ds(ci, 1), :],
                                      ec_v.at[pl.ds(i, 1), :], sem).start()
            return 0

        lax.fori_loop(0, _CHUNK // _LANES, issue, 0)
        pltpu.make_async_copy(lt_hbm.at[pl.ds(0, _CHUNK), :], el_v, sem).wait()
        pltpu.make_async_copy(ct_hbm.at[pl.ds(0, _CHUNK), :], ec_v, sem).wait()
        pltpu.sync_copy(el_v, el_hbm.at[pl.ds(base + off, _CHUNK)])
        pltpu.sync_copy(ec_v, ec_hbm.at[pl.ds(base + off, _CHUNK)])


@jax.jit
def _sc_gather(label_table, cat_table, label_idx, category_idx):
    mesh = plsc.VectorSubcoreMesh(core_axis_name="c", subcore_axis_name="s")
    out_type = [
        jax.ShapeDtypeStruct((BATCH, EMBED_DIM), jnp.float32),
        jax.ShapeDtypeStruct((BATCH, EMBED_DIM), jnp.float32),
    ]
    scratch = [
        pltpu.VMEM((_B_PER_W,), jnp.int32),
        pltpu.VMEM((_B_PER_W,), jnp.int32),
        pltpu.VMEM((_CHUNK, EMBED_DIM), jnp.float32),
        pltpu.VMEM((_CHUNK, EMBED_DIM), jnp.float32),
        pltpu.SemaphoreType.DMA,
    ]
    fn = pl.kernel(_gather_body, out_type=out_type, mesh=mesh,
                   scratch_types=scratch,
                   compiler_params=pltpu.CompilerParams(
                       needs_layout_passes=False))
    return fn(label_table, cat_table, label_idx, category_idx)


def _mlp_body(x1_ref, x2_ref, w1a_ref, w1b_ref, b1_ref, w2_ref, b2_ref,
              w3_ref, b3_ref, o_ref):
    x1 = x1_ref[...]
    x2 = x2_ref[...]
    h = jnp.dot(x1, w1a_ref[...], preferred_element_type=jnp.float32)
    h += jnp.dot(x2, w1b_ref[...], preferred_element_type=jnp.float32)
    h = jnp.maximum(h + b1_ref[...], 0.0)
    h = jnp.dot(h, w2_ref[...], preferred_element_type=jnp.float32)
    h = jnp.maximum(h + b2_ref[...], 0.0)
    o_ref[...] = jnp.dot(h, w3_ref[...],
                         preferred_element_type=jnp.float32) + b3_ref[...]


@functools.partial(jax.jit, static_argnames=("bm",))
def _tc_mlp(e_label, e_cat, W1a, W1b, b1, W2, b2, W3, b3, bm=2048):
    grid = (BATCH // bm,)
    full = lambda shape: pl.BlockSpec(shape, lambda i: (0, 0))
    return pl.pallas_call(
        _mlp_body,
        grid=grid,
        in_specs=[
            pl.BlockSpec((bm, EMBED_DIM), lambda i: (i, 0)),
            pl.BlockSpec((bm, EMBED_DIM), lambda i: (i, 0)),
            full(W1a.shape),
            full(W1b.shape),
            full(b1.shape),
            full(W2.shape),
            full(b2.shape),
            full(W3.shape),
            full(b3.shape),
        ],
        out_specs=pl.BlockSpec((bm, 2), lambda i: (i, 0)),
        out_shape=jax.ShapeDtypeStruct((BATCH, 2), jnp.float32),
    )(e_label, e_cat, W1a, W1b, b1, W2, b2, W3, b3)


def kernel(label_idx, category_idx, label_table, cat_table,
           W1, b1, W2, b2, W3, b3):
    li = label_idx.astype(jnp.int32)
    ci = category_idx.astype(jnp.int32)
    e_label, e_cat = _sc_gather(label_table, cat_table, li, ci)
    W1a = W1[:EMBED_DIM]
    W1b = W1[EMBED_DIM:]
    return _tc_mlp(e_label, e_cat, W1a, W1b, b1.reshape(1, -1), W2,
                   b2.reshape(1, -1), W3, b3.reshape(1, -1))
